# Initial kernel scaffold; baseline (speedup 1.0000x reference)
import jax
import jax.numpy as jnp
from jax.experimental import pallas as pl


def kernel(pred, target):
    raise NotImplementedError("write your pallas kernel here")


# speedup vs baseline: 17.1364x; 17.1364x over previous
import functools

import jax
import jax.numpy as jnp
import numpy as np
from jax import lax
from jax.experimental import pallas as pl
from jax.experimental.pallas import tpu as pltpu
from jax.experimental.pallas import tpu_sc as plsc

_N = 8388608
_NC = 2
_NS = 16
_NW = _NC * _NS
_L = 16

_EDGE10 = float(np.float32(1.0) + np.float32(1e-6))
_MOMENTUM = 0.5


def _make_sc_partials(n_total, chunk, interpret=False):
    per_w = n_total // _NW
    chunks = per_w // chunk
    vecs = chunk // _L
    assert per_w * _NW == n_total and chunks * chunk == per_w and vecs * _L == chunk

    def body(pred_hbm, target_hbm, out_hbm, pbuf, tbuf, acc_v):
        wid = lax.axis_index("s") * _NC + lax.axis_index("c")
        base = wid * per_w

        zeros = jnp.zeros((_L,), jnp.float32)
        ones = jnp.ones((_L,), jnp.float32)
        neg_big = jnp.full((_L,), -1e30, jnp.float32)

        def chunk_body(c, carry):
            M, S, B, A, Bgt, Agt, cnt = carry
            start = base + c * chunk
            pltpu.sync_copy(pred_hbm.at[pl.ds(start, chunk)], pbuf)
            pltpu.sync_copy(target_hbm.at[pl.ds(start, chunk)], tbuf)

            def pass1(i, c1):
                cm, B, A, Bgt, Agt, cnt = c1
                v = pbuf[pl.ds(i * _L, _L)]
                t = tbuf[pl.ds(i * _L, _L)]
                cm = jnp.maximum(cm, v)
                tp = t * v
                B = B + t
                A = A + tp
                gt = t > v
                Bgt = Bgt + jnp.where(gt, t, zeros)
                Agt = Agt + jnp.where(gt, tp, zeros)
                sg = 1.0 / (1.0 + jnp.exp(-v))
                g = jnp.abs(sg - t)
                cnt = cnt + jnp.where(g <= _EDGE10, ones, zeros)
                return (cm, B, A, Bgt, Agt, cnt)

            cm, B, A, Bgt, Agt, cnt = lax.fori_loop(
                0, vecs, pass1, (neg_big, B, A, Bgt, Agt, cnt), unroll=4)

            Mn = jnp.maximum(M, cm)
            S = S * jnp.exp(M - Mn)

            def pass2(i, s):
                v = pbuf[pl.ds(i * _L, _L)]
                return s + jnp.exp(v - Mn)

            S = lax.fori_loop(0, vecs, pass2, S, unroll=4)
            return (Mn, S, B, A, Bgt, Agt, cnt)

        init = (neg_big, zeros, zeros, zeros, zeros, zeros, zeros)
        M, S, B, A, Bgt, Agt, cnt = lax.fori_loop(0, chunks, chunk_body, init)

        acc_v[0] = M
        acc_v[1] = S
        acc_v[2] = B
        acc_v[3] = A
        acc_v[4] = Bgt
        acc_v[5] = Agt
        acc_v[6] = cnt
        acc_v[7] = zeros
        pltpu.sync_copy(acc_v, out_hbm.at[wid])

    return pl.kernel(
        body,
        out_type=jax.ShapeDtypeStruct((_NW, 8, _L), jnp.float32),
        mesh=plsc.VectorSubcoreMesh(
            core_axis_name="c", subcore_axis_name="s",
            num_cores=_NC, num_subcores=_NS),
        scratch_types=[
            pltpu.VMEM((chunk,), jnp.float32),
            pltpu.VMEM((chunk,), jnp.float32),
            pltpu.VMEM((8, _L), jnp.float32),
        ],
        interpret=interpret,
    )


def _combine(parts, n_total):
    M = parts[:, 0, :]
    S = parts[:, 1, :]
    B = parts[:, 2, :]
    A = parts[:, 3, :]
    Bgt = parts[:, 4, :]
    Agt = parts[:, 5, :]
    cnt = parts[:, 6, :]
    Mg = jnp.max(M)
    S_tot = jnp.sum(S * jnp.exp(M - Mg))
    logZ = Mg + jnp.log(S_tot)
    acc_sum = (1.0 - _MOMENTUM) * jnp.sum(cnt)
    total_neg = jnp.float32(n_total) - acc_sum
    total_pos = jnp.maximum(acc_sum, 1.0)
    ratio = total_neg / total_pos
    r1 = ratio - 1.0
    return -(jnp.sum(A) + r1 * jnp.sum(Agt)) + logZ * (jnp.sum(B) + r1 * jnp.sum(Bgt))


@functools.lru_cache(maxsize=None)
def _sc_partials():
    return _make_sc_partials(_N, 8192)


def kernel(pred, target):
    parts = _sc_partials()(pred, target)
    return _combine(parts, _N)


# speedup vs baseline: 25.9336x; 1.5134x over previous
import functools

import jax
import jax.numpy as jnp
import numpy as np
from jax import lax
from jax.experimental import pallas as pl
from jax.experimental.pallas import tpu as pltpu
from jax.experimental.pallas import tpu_sc as plsc

_N = 8388608
_NC = 2
_NS = 16
_NW = _NC * _NS
_L = 16

_EDGE10 = float(np.float32(1.0) + np.float32(1e-6))
_MOMENTUM = 0.5


def _make_sc_partials(n_total, chunk, interpret=False):
    per_w = n_total // _NW
    chunks = per_w // chunk
    vecs = chunk // _L
    assert per_w * _NW == n_total and chunks * chunk == per_w and vecs * _L == chunk

    assert chunks % 2 == 0

    def body(pred_hbm, target_hbm, out_hbm, pa, ta, pb, tb, acc_v, sem_a, sem_b):
        wid = lax.axis_index("s") * _NC + lax.axis_index("c")
        base = wid * per_w

        zeros = jnp.zeros((_L,), jnp.float32)
        ones = jnp.ones((_L,), jnp.float32)
        neg_big = jnp.full((_L,), -1e30, jnp.float32)

        def start(bp, bt, sem, c):
            s0 = base + c * chunk
            pltpu.async_copy(pred_hbm.at[pl.ds(s0, chunk)], bp, sem)
            pltpu.async_copy(target_hbm.at[pl.ds(s0, chunk)], bt, sem)

        def wait(bp, bt, sem, c):
            s0 = base + c * chunk
            pltpu.make_async_copy(pred_hbm.at[pl.ds(s0, chunk)], bp, sem).wait()
            pltpu.make_async_copy(target_hbm.at[pl.ds(s0, chunk)], bt, sem).wait()

        def compute(pbuf, tbuf, carry):
            M, S, B, A, Bgt, Agt, cnt = carry

            def pass1(i, c1):
                cm, B, A, Bgt, Agt, cnt = c1
                v = pbuf[pl.ds(i * _L, _L)]
                t = tbuf[pl.ds(i * _L, _L)]
                cm = jnp.maximum(cm, v)
                tp = t * v
                B = B + t
                A = A + tp
                gt = t > v
                Bgt = Bgt + jnp.where(gt, t, zeros)
                Agt = Agt + jnp.where(gt, tp, zeros)
                sg = 1.0 / (1.0 + jnp.exp(-v))
                g = jnp.abs(sg - t)
                cnt = cnt + jnp.where(g <= _EDGE10, ones, zeros)
                return (cm, B, A, Bgt, Agt, cnt)

            cm, B, A, Bgt, Agt, cnt = lax.fori_loop(
                0, vecs, pass1, (neg_big, B, A, Bgt, Agt, cnt), unroll=4)

            Mn = jnp.maximum(M, cm)
            S = S * jnp.exp(M - Mn)

            def pass2(i, s):
                v = pbuf[pl.ds(i * _L, _L)]
                return s + jnp.exp(v - Mn)

            S = lax.fori_loop(0, vecs, pass2, S, unroll=4)
            return (Mn, S, B, A, Bgt, Agt, cnt)

        start(pa, ta, sem_a, 0)

        def pair_body(i, carry):
            c0 = 2 * i
            start(pb, tb, sem_b, c0 + 1)
            wait(pa, ta, sem_a, c0)
            carry = compute(pa, ta, carry)

            @pl.when(c0 + 2 < chunks)
            def _():
                start(pa, ta, sem_a, c0 + 2)

            wait(pb, tb, sem_b, c0 + 1)
            carry = compute(pb, tb, carry)
            return carry

        init = (neg_big, zeros, zeros, zeros, zeros, zeros, zeros)
        M, S, B, A, Bgt, Agt, cnt = lax.fori_loop(0, chunks // 2, pair_body, init)

        acc_v[0] = M
        acc_v[1] = S
        acc_v[2] = B
        acc_v[3] = A
        acc_v[4] = Bgt
        acc_v[5] = Agt
        acc_v[6] = cnt
        acc_v[7] = zeros
        pltpu.sync_copy(acc_v, out_hbm.at[wid])

    return pl.kernel(
        body,
        out_type=jax.ShapeDtypeStruct((_NW, 8, _L), jnp.float32),
        mesh=plsc.VectorSubcoreMesh(
            core_axis_name="c", subcore_axis_name="s",
            num_cores=_NC, num_subcores=_NS),
        scratch_types=[
            pltpu.VMEM((chunk,), jnp.float32),
            pltpu.VMEM((chunk,), jnp.float32),
            pltpu.VMEM((chunk,), jnp.float32),
            pltpu.VMEM((chunk,), jnp.float32),
            pltpu.VMEM((8, _L), jnp.float32),
            pltpu.SemaphoreType.DMA,
            pltpu.SemaphoreType.DMA,
        ],
        interpret=interpret,
    )


def _combine(parts, n_total):
    M = parts[:, 0, :]
    S = parts[:, 1, :]
    B = parts[:, 2, :]
    A = parts[:, 3, :]
    Bgt = parts[:, 4, :]
    Agt = parts[:, 5, :]
    cnt = parts[:, 6, :]
    Mg = jnp.max(M)
    S_tot = jnp.sum(S * jnp.exp(M - Mg))
    logZ = Mg + jnp.log(S_tot)
    acc_sum = (1.0 - _MOMENTUM) * jnp.sum(cnt)
    total_neg = jnp.float32(n_total) - acc_sum
    total_pos = jnp.maximum(acc_sum, 1.0)
    ratio = total_neg / total_pos
    r1 = ratio - 1.0
    return -(jnp.sum(A) + r1 * jnp.sum(Agt)) + logZ * (jnp.sum(B) + r1 * jnp.sum(Bgt))


@functools.lru_cache(maxsize=None)
def _sc_partials():
    return _make_sc_partials(_N, 16384)


def kernel(pred, target):
    parts = _sc_partials()(pred, target)
    return _combine(parts, _N)
